# Initial kernel scaffold; baseline (speedup 1.0000x reference)
import jax
import jax.numpy as jnp
from jax.experimental import pallas as pl


def kernel(x_doc2vec, x_kmer, x_rnafm, x_sim, edge_index, W_fuse, b_fuse, W1, b1, W2, b2):
    raise NotImplementedError("write your pallas kernel here")


# speedup vs baseline: 5.6842x; 5.6842x over previous
import functools

import jax
import jax.numpy as jnp
from jax import lax
from jax.experimental import pallas as pl
from jax.experimental.pallas import tpu as pltpu
from jax.experimental.pallas import tpu_sc as plsc

N = 10000
E = 160000
D = 256
DH = 128
NC = 2
NS = 16
ACC_ROWS = 10240
TRASH = N
E_PAD = 161792
CH = 128
EPT = E_PAD // NS
NCHUNK = EPT // CH
DEG_CH = 64
EPT_DEG = E_PAD // (NC * NS)
NCHUNK_DEG = EPT_DEG // DEG_CH
ZROWS = ACC_ROWS // NS

_mesh = plsc.VectorSubcoreMesh(core_axis_name="c", subcore_axis_name="s")


@functools.partial(
    pl.kernel,
    out_type=jax.ShapeDtypeStruct((NC, ACC_ROWS, 16), jnp.float32),
    mesh=_mesh,
    scratch_types=[
        pltpu.VMEM((DEG_CH,), jnp.int32),
        pltpu.VMEM((DEG_CH, 16), jnp.float32),
        pltpu.VMEM_SHARED((ACC_ROWS, 16), jnp.float32),
    ],
)
def _sc_degree(dst_hbm, ones_hbm, zeros_hbm, out_hbm, dstv, onesv, acc):
    c = lax.axis_index("c")
    s = lax.axis_index("s")
    pltpu.sync_copy(zeros_hbm, acc.at[pl.ds(s * ZROWS, ZROWS)])
    pltpu.sync_copy(ones_hbm, onesv)
    plsc.subcore_barrier()

    base = (c * NS + s) * EPT_DEG

    @pl.loop(0, NCHUNK_DEG)
    def _(k):
        pltpu.sync_copy(dst_hbm.at[pl.ds(base + k * DEG_CH, DEG_CH)], dstv)
        pltpu.sync_copy(onesv, acc.at[dstv], add=True)

    plsc.subcore_barrier()
    pltpu.sync_copy(acc.at[pl.ds(s * ZROWS, ZROWS)],
                    out_hbm.at[c, pl.ds(s * ZROWS, ZROWS)])


@functools.partial(
    pl.kernel,
    out_type=jax.ShapeDtypeStruct((NC, ACC_ROWS, DH), jnp.float32),
    mesh=_mesh,
    scratch_types=[
        pltpu.VMEM((CH,), jnp.int32),
        pltpu.VMEM((CH,), jnp.int32),
        pltpu.VMEM((CH, DH), jnp.float32),
        pltpu.VMEM_SHARED((ACC_ROWS, DH), jnp.float32),
    ],
)
def _sc_aggregate(tbl_hbm, src2_hbm, dst_hbm, zeros_hbm, out_hbm,
                  srcv, dstv, rowsv, acc):
    c = lax.axis_index("c")
    s = lax.axis_index("s")
    pltpu.sync_copy(zeros_hbm, acc.at[pl.ds(s * ZROWS, ZROWS)])
    plsc.subcore_barrier()

    base = s * EPT

    @pl.loop(0, NCHUNK)
    def _(k):
        off = base + k * CH
        pltpu.sync_copy(src2_hbm.at[c, pl.ds(off, CH)], srcv)
        pltpu.sync_copy(dst_hbm.at[pl.ds(off, CH)], dstv)
        pltpu.sync_copy(tbl_hbm.at[srcv], rowsv)
        pltpu.sync_copy(rowsv, acc.at[dstv], add=True)

    plsc.subcore_barrier()
    pltpu.sync_copy(acc.at[pl.ds(s * ZROWS, ZROWS)],
                    out_hbm.at[c, pl.ds(s * ZROWS, ZROWS)])


RB = 1000
GRID = N // RB


def _fuse_body(x1, x2, x3, x4, wf, bf, h_out):
    acc = jnp.dot(x1[...], wf[0:128, :], preferred_element_type=jnp.float32,
                  precision=lax.Precision.HIGHEST)
    acc += jnp.dot(x2[...], wf[128:384, :], preferred_element_type=jnp.float32,
                   precision=lax.Precision.HIGHEST)
    acc += jnp.dot(x3[...], wf[384:1024, :], preferred_element_type=jnp.float32,
                   precision=lax.Precision.HIGHEST)
    acc += jnp.dot(x4[...], wf[1024:1280, :], preferred_element_type=jnp.float32,
                   precision=lax.Precision.HIGHEST)
    h_out[...] = jnp.maximum(acc + bf[...], 0.0)


def _tc_fuse(x1, x2, x3, x4, wf, bf):
    return pl.pallas_call(
        _fuse_body,
        grid=(GRID,),
        in_specs=[
            pl.BlockSpec((RB, 128), lambda i: (i, 0)),
            pl.BlockSpec((RB, 256), lambda i: (i, 0)),
            pl.BlockSpec((RB, 640), lambda i: (i, 0)),
            pl.BlockSpec((RB, 256), lambda i: (i, 0)),
            pl.BlockSpec((1280, 256), lambda i: (0, 0)),
            pl.BlockSpec((1, 256), lambda i: (0, 0)),
        ],
        out_specs=pl.BlockSpec((RB, D), lambda i: (i, 0)),
        out_shape=jax.ShapeDtypeStruct((N, D), jnp.float32),
    )(x1, x2, x3, x4, wf, bf)


def _deg_terms(c0, c1):
    deg = c0[:, 0:1] + c1[:, 0:1] + 1.0
    return lax.rsqrt(deg), 1.0 / deg


def _pre_body(h, c0, c1, hp_out):
    r, _ = _deg_terms(c0[...], c1[...])
    hs = h[...] * r
    hp_out[...] = jnp.stack([hs[:, 0:DH], hs[:, DH:D]], axis=0)


def _tc_pre(h, c0, c1):
    return pl.pallas_call(
        _pre_body,
        grid=(GRID,),
        in_specs=[
            pl.BlockSpec((RB, D), lambda i: (i, 0)),
            pl.BlockSpec((RB, 16), lambda i: (i, 0)),
            pl.BlockSpec((RB, 16), lambda i: (i, 0)),
        ],
        out_specs=pl.BlockSpec((NC, RB, DH), lambda i: (0, i, 0)),
        out_shape=jax.ShapeDtypeStruct((NC, N, DH), jnp.float32),
    )(h, c0, c1)


def _post_body(relu, emit_pre, agg, h, c0, c1, w, b, *outs):
    r, inv = _deg_terms(c0[...], c1[...])
    aggcat = jnp.concatenate([agg[0], agg[1]], axis=1)
    m = aggcat * r + h[...] * inv
    z = jnp.dot(m, w[...], preferred_element_type=jnp.float32,
                precision=lax.Precision.HIGHEST) + b[...]
    if relu:
        z = jnp.maximum(z, 0.0)
    outs[0][...] = z
    if emit_pre:
        zs = z * r
        outs[1][...] = jnp.stack([zs[:, 0:DH], zs[:, DH:D]], axis=0)


def _tc_post(agg, h, c0, c1, w, b, relu, emit_pre):
    out_shape = [jax.ShapeDtypeStruct((N, D), jnp.float32)]
    out_specs = [pl.BlockSpec((RB, D), lambda i: (i, 0))]
    if emit_pre:
        out_shape.append(jax.ShapeDtypeStruct((NC, N, DH), jnp.float32))
        out_specs.append(pl.BlockSpec((NC, RB, DH), lambda i: (0, i, 0)))
    return pl.pallas_call(
        functools.partial(_post_body, relu, emit_pre),
        grid=(GRID,),
        in_specs=[
            pl.BlockSpec((NC, RB, DH), lambda i: (0, i, 0)),
            pl.BlockSpec((RB, D), lambda i: (i, 0)),
            pl.BlockSpec((RB, 16), lambda i: (i, 0)),
            pl.BlockSpec((RB, 16), lambda i: (i, 0)),
            pl.BlockSpec((D, D), lambda i: (0, 0)),
            pl.BlockSpec((1, 256), lambda i: (0, 0)),
        ],
        out_specs=out_specs,
        out_shape=out_shape,
    )(agg, h, c0, c1, w, b)


def kernel(x_doc2vec, x_kmer, x_rnafm, x_sim, edge_index,
           W_fuse, b_fuse, W1, b1, W2, b2):
    src = edge_index[0]
    dst = edge_index[1]
    pad = E_PAD - E
    src_p = jnp.concatenate([src, jnp.zeros((pad,), jnp.int32)])
    dst_p = jnp.concatenate([dst, jnp.full((pad,), TRASH, jnp.int32)])
    src2 = jnp.stack([src_p, src_p + N])

    ones16 = jnp.ones((DEG_CH, 16), jnp.float32)
    zeros16 = jnp.zeros((ZROWS, 16), jnp.float32)
    zerosd = jnp.zeros((ZROWS, DH), jnp.float32)
    bf = b_fuse.reshape(1, D)
    b1r = b1.reshape(1, D)
    b2r = b2.reshape(1, D)

    cnt = _sc_degree(dst_p, ones16, zeros16)
    c0, c1 = cnt[0], cnt[1]

    h = _tc_fuse(x_doc2vec, x_kmer, x_rnafm, x_sim, W_fuse, bf)
    hp = _tc_pre(h, c0, c1).reshape(NC * N, DH)

    agg1 = _sc_aggregate(hp, src2, dst_p, zerosd)
    h1, hp1 = _tc_post(agg1, h, c0, c1, W1, b1r, relu=True, emit_pre=True)

    agg2 = _sc_aggregate(hp1.reshape(NC * N, DH), src2, dst_p, zerosd)
    out = _tc_post(agg2, h1, c0, c1, W2, b2r, relu=False, emit_pre=False)
    return out[0]
